# Initial kernel scaffold; baseline (speedup 1.0000x reference)
import jax
import jax.numpy as jnp
from jax.experimental import pallas as pl


def kernel(x, indices, distances, eps, k_param):
    raise NotImplementedError("write your pallas kernel here")


# speedup vs baseline: 346.0756x; 346.0756x over previous
import functools

import jax
import jax.numpy as jnp
from jax import lax
from jax.experimental import pallas as pl
from jax.experimental.pallas import tpu as pltpu
from jax.experimental.pallas import tpu_sc as plsc

_NU = 1
_L = 16
_NW = 32


def _prep_body(nie_ref, dist_ref, x_ref, packed_ref, sxx_ref):
    i = pl.program_id(0)
    e = jnp.exp(dist_ref[...] * nie_ref[0, 0]).astype(jnp.bfloat16)
    ones8 = jnp.ones((8, e.shape[1]), jnp.bfloat16)
    s8 = lax.dot_general(ones8, e, (((1,), (1,)), ((), ())),
                         preferred_element_type=jnp.float32)
    dinv = 1.0 / s8[0:1, :]
    xb = x_ref[0, :, :]

    def bf16_bits(v):
        u = lax.bitcast_convert_type(v, jnp.uint32)
        r = u + jnp.uint32(0x7FFF) + ((u >> 16) & jnp.uint32(1))
        return r & jnp.uint32(0xFFFF0000)

    packed_ref[0, :, :] = lax.bitcast_convert_type(
        bf16_bits(dinv) | (bf16_bits(xb) >> 16), jnp.int32)

    @pl.when(i == 0)
    def _():
        sxx_ref[0, 0] = 0.0

    sxx_ref[0, 0] += jnp.sum(xb * xb)


def _edge_body(n, kk, dist_hbm, idx_hbm, packed_hbm, nie_hbm, out_hbm,
               table_v, dist_v, idx_v, misc_v, sem_d, sem_i):
    groups = n // _L
    base_g = groups // _NW
    extra = groups % _NW
    wid = lax.axis_index("c") * _L + lax.axis_index("s")
    g0 = wid * base_g + jnp.minimum(wid, extra)
    ng = base_g + (wid < extra).astype(jnp.int32)

    pltpu.sync_copy(packed_hbm, table_v)
    pltpu.sync_copy(nie_hbm, misc_v)
    nie = misc_v[...]

    lanes = jnp.arange(_L, dtype=jnp.int32)

    def issue(g, p):
        pltpu.async_copy(dist_hbm.at[g], dist_v.at[p], sem_d)
        pltpu.async_copy(idx_hbm.at[g], idx_v.at[p], sem_i)

    def wait(p):
        pltpu.make_async_copy(dist_hbm.at[0], dist_v.at[p], sem_d).wait()
        pltpu.make_async_copy(idx_hbm.at[0], idx_v.at[p], sem_i).wait()

    issue(g0, jnp.int32(0))

    def body(k, acc):
        p = lax.rem(k, 2)
        wait(p)

        @pl.when(k + 1 < ng)
        def _():
            issue(g0 + k + 1, 1 - p)

        pvec = jnp.full((_L,), p, dtype=jnp.int32)
        g = g0 + k
        rows = g * _L + lanes
        own = plsc.load_gather(table_v, [rows])
        xi = lax.bitcast_convert_type(lax.shift_left(own, 16), jnp.float32)

        num = jnp.zeros((_L,), jnp.float32)
        den = jnp.zeros((_L,), jnp.float32)
        for j in range(kk):
            jvec = jnp.full((_L,), j, dtype=jnp.int32)
            d = plsc.load_gather(dist_v, [pvec, lanes, jvec])
            ii = plsc.load_gather(idx_v, [pvec, lanes, jvec])
            pk = plsc.load_gather(table_v, [ii])
            dg = lax.bitcast_convert_type(
                lax.bitwise_and(pk, jnp.int32(-65536)), jnp.float32)
            xg = lax.bitcast_convert_type(lax.shift_left(pk, 16), jnp.float32)
            u = jnp.exp(d * nie) * dg
            den = den + u
            num = num + u * xg
        return acc + xi * num / den

    acc = lax.fori_loop(0, ng, body, jnp.zeros((_L,), jnp.float32))
    misc_v[...] = acc
    pltpu.sync_copy(misc_v, out_hbm.at[wid])


def kernel(x, indices, distances, eps, k_param):
    n, kk = distances.shape
    groups = n // _L
    nie = (-1.0 / eps).astype(jnp.float32)

    bp = 2000
    packed, sxx = pl.pallas_call(
        _prep_body,
        grid=(n // bp,),
        in_specs=[
            pl.BlockSpec(memory_space=pltpu.SMEM),
            pl.BlockSpec((bp, kk), lambda i: (i, 0)),
            pl.BlockSpec((1, 1, bp), lambda i: (i, 0, 0)),
        ],
        out_specs=[
            pl.BlockSpec((1, 1, bp), lambda i: (i, 0, 0)),
            pl.BlockSpec(memory_space=pltpu.SMEM),
        ],
        out_shape=[
            jax.ShapeDtypeStruct((n // bp, 1, bp), jnp.int32),
            jax.ShapeDtypeStruct((1, 1), jnp.float32),
        ],
    )(nie.reshape(1, 1), distances, x.reshape(n // bp, 1, bp))
    packed = packed.reshape(n)

    dist3 = distances.reshape(groups, _L, kk)
    idx3 = indices.astype(jnp.int32).reshape(groups, _L, kk)
    nie16 = jnp.full((_L,), nie, dtype=jnp.float32)

    mesh = plsc.VectorSubcoreMesh(core_axis_name="c", subcore_axis_name="s")
    edge = functools.partial(
        pl.kernel,
        mesh=mesh,
        compiler_params=pltpu.CompilerParams(needs_layout_passes=False),
        out_type=jax.ShapeDtypeStruct((_NW, _L), jnp.float32),
        scratch_types=[
            pltpu.VMEM((n,), jnp.int32),
            pltpu.VMEM((2, _L, kk), jnp.float32),
            pltpu.VMEM((2, _L, kk), jnp.int32),
            pltpu.VMEM((_L,), jnp.float32),
            pltpu.SemaphoreType.DMA,
            pltpu.SemaphoreType.DMA,
        ],
    )(functools.partial(_edge_body, n, kk))
    parts = edge(dist3, idx3, packed, nie16)

    c0 = 4.0 / eps + 2.0 * _NU / (k_param * k_param) + 10.0
    out = c0 * sxx[0, 0] - (4.0 / eps) * jnp.sum(parts)
    return out.astype(jnp.float32)
